# baseline (speedup 1.0000x reference)
import functools

import jax
import jax.numpy as jnp
from jax import lax
from jax.experimental import pallas as pl
from jax.experimental.pallas import tpu as pltpu
from jax.experimental.pallas import tpu_sc as plsc

_INFO = plsc.get_sparse_core_info()
_NC = _INFO.num_cores
_NS = _INFO.num_subcores
_L = _INFO.num_lanes
_NW = _NC * _NS

_K = 13
_D = 64
_C = 128


def _argmax_group(in_v, pos_base):
    flat = (pos_base + lax.iota(jnp.int32, _L)) * _K
    best_v = plsc.load_gather(in_v, [flat])
    best_i = jnp.zeros((_L,), jnp.int32)
    for k in range(1, _K):
        kv = jnp.full((_L,), k, jnp.int32)
        v = plsc.load_gather(in_v, [flat + k])
        upd = v > best_v
        best_v = jnp.where(upd, v, best_v)
        best_i = jnp.where(upd, kv, best_i)
    return best_i


def _body(x_hbm, tab_hbm, out_hbm, in_v, idx_v, rows_v, sem):
    wid = lax.axis_index("s") * _NC + lax.axis_index("c")
    n = x_hbm.shape[0] // _K
    per_w = n // _NW
    chunks = per_w // _C

    def chunk(t, _):
        base = pl.multiple_of(wid * per_w + t * _C, _C)
        pltpu.sync_copy(x_hbm.at[pl.ds(base * _K, _C * _K)], in_v)
        for g in range(_C // _L):
            best_i = _argmax_group(in_v, g * _L)
            idx_v[pl.ds(g * _L, _L)] = best_i
        pltpu.async_copy(tab_hbm.at[idx_v], rows_v, sem).wait()
        pltpu.sync_copy(rows_v, out_hbm.at[pl.ds(base, _C)])
        return ()

    lax.fori_loop(0, chunks, chunk, (), unroll=False)


@functools.partial(jax.jit, static_argnames=())
def kernel(one_hot_pieces, piece_embedding):
    b = one_hot_pieces.shape[0]
    n = b * 64
    x = one_hot_pieces.reshape(n * _K)

    mesh = plsc.VectorSubcoreMesh(core_axis_name="c", subcore_axis_name="s")
    run = pl.kernel(
        _body,
        mesh=mesh,
        out_type=jax.ShapeDtypeStruct((n, _D), jnp.float32),
        scratch_types=[
            pltpu.VMEM((_C * _K,), jnp.float32),
            pltpu.VMEM((_C,), jnp.int32),
            pltpu.VMEM((_C, _D), jnp.float32),
            pltpu.SemaphoreType.DMA,
        ],
        compiler_params=pltpu.CompilerParams(
            needs_layout_passes=False, use_tc_tiling_on_sc=False
        ),
    )
    out = run(x, piece_embedding)
    return out.reshape(b, 64, _D)


# speedup vs baseline: 1.3229x; 1.3229x over previous
import functools

import jax
import jax.numpy as jnp
from jax import lax
from jax.experimental import pallas as pl
from jax.experimental.pallas import tpu as pltpu
from jax.experimental.pallas import tpu_sc as plsc

_INFO = plsc.get_sparse_core_info()
_NC = _INFO.num_cores
_NS = _INFO.num_subcores
_L = _INFO.num_lanes
_NW = _NC * _NS

_K = 13
_D = 64
_C = 512
_NBUF = 2


def _do_group(in_v, tab_v, rows_v, g, iota, iota_d):
    flat = (g * (_L * _K)) + iota * _K
    best_v = plsc.load_gather(in_v, [flat])
    best_i = jnp.zeros((_L,), jnp.int32)
    for k in range(1, _K):
        kv = jnp.full((_L,), k, jnp.int32)
        v = plsc.load_gather(in_v, [flat + k])
        upd = v > best_v
        best_v = jnp.where(upd, v, best_v)
        best_i = jnp.where(upd, kv, best_i)
    tbase = best_i * _D
    obase = g * (_L * _D) + iota_d
    for j in range(_D):
        v = plsc.load_gather(tab_v, [tbase + j])
        plsc.store_scatter(rows_v, [obase + j], v)


def _body(x_hbm, tab_hbm, out_hbm, tab_v, in_v, rows_v, sems):
    wid = lax.axis_index("s") * _NC + lax.axis_index("c")
    n = x_hbm.shape[0] // _K
    per_w = n // _NW
    chunks = per_w // _C
    w_base = wid * per_w

    pltpu.sync_copy(tab_hbm, tab_v)
    iota = lax.iota(jnp.int32, _L)
    iota_d = iota * _D

    for b in range(_NBUF):
        pltpu.async_copy(
            x_hbm.at[pl.ds((w_base + b * _C) * _K, _C * _K)],
            in_v.at[b],
            sems.at[b],
        )

    def step(t2, _):
        for b in range(_NBUF):
            t = t2 * _NBUF + b
            base = pl.multiple_of(w_base + t * _C, _C)
            pltpu.make_async_copy(
                x_hbm.at[pl.ds(0, _C * _K)], in_v.at[b], sems.at[b]
            ).wait()
            @pl.when(t2 > 0)
            def _():
                pltpu.make_async_copy(
                    rows_v.at[b],
                    out_hbm.at[pl.ds(0, _C * _D)],
                    sems.at[_NBUF + b],
                ).wait()

            def group(g, _):
                _do_group(in_v.at[b], tab_v, rows_v.at[b], g, iota, iota_d)
                return ()

            lax.fori_loop(0, _C // _L, group, (), unroll=False)
            pltpu.async_copy(
                rows_v.at[b],
                out_hbm.at[pl.ds(base * _D, _C * _D)],
                sems.at[_NBUF + b],
            )
            @pl.when(t2 * _NBUF + b + _NBUF < chunks)
            def _():
                pltpu.async_copy(
                    x_hbm.at[pl.ds((base + _NBUF * _C) * _K, _C * _K)],
                    in_v.at[b],
                    sems.at[b],
                )

        return ()

    lax.fori_loop(0, chunks // _NBUF, step, (), unroll=False)
    for b in range(_NBUF):
        pltpu.make_async_copy(
            rows_v.at[b], out_hbm.at[pl.ds(0, _C * _D)], sems.at[_NBUF + b]
        ).wait()


@functools.partial(jax.jit, static_argnames=())
def kernel(one_hot_pieces, piece_embedding):
    b = one_hot_pieces.shape[0]
    n = b * 64
    x = one_hot_pieces.reshape(n * _K)
    tab = piece_embedding.reshape(_K * _D)

    mesh = plsc.VectorSubcoreMesh(core_axis_name="c", subcore_axis_name="s")
    run = pl.kernel(
        _body,
        mesh=mesh,
        out_type=jax.ShapeDtypeStruct((n * _D,), jnp.float32),
        scratch_types=[
            pltpu.VMEM((_K * _D,), jnp.float32),
            pltpu.VMEM((_NBUF, _C * _K), jnp.float32),
            pltpu.VMEM((_NBUF, _C * _D), jnp.float32),
            pltpu.SemaphoreType.DMA((2 * _NBUF,)),
        ],
        compiler_params=pltpu.CompilerParams(
            needs_layout_passes=False, use_tc_tiling_on_sc=False
        ),
    )
    out = run(x, tab)
    return out.reshape(b, 64, _D)
